# Initial kernel scaffold; baseline (speedup 1.0000x reference)
import jax
import jax.numpy as jnp
from jax.experimental import pallas as pl


def kernel(x, W_q, W_o, b_o, keys, values):
    raise NotImplementedError("write your pallas kernel here")


# speedup vs baseline: 1.9266x; 1.9266x over previous
import functools

import jax
import jax.numpy as jnp
from jax import lax
from jax.experimental import pallas as pl
from jax.experimental.pallas import tpu as pltpu
from jax.experimental.pallas import tpu_sc as plsc

_DIM = 2048
_HEADS = 8
_NK = 128
_TK = 16
_DH = _DIM // _HEADS

_TBLK = 512


def _proj_score_body(x_ref, wq_ref, k_ref, d_ref):
    xb = x_ref[0]
    wq = wq_ref[0]
    kk = k_ref[0]
    q = lax.dot_general(xb, wq, (((1,), (1,)), ((), ())),
                        preferred_element_type=jnp.float32)
    d = lax.dot_general(q, kk, (((1,), (1,)), ((), ())),
                        preferred_element_type=jnp.float32)
    d_ref[0, 0] = d


def _proj_score(x, wq_r, k_r):
    b, t, e = x.shape
    grid = (b, t // _TBLK, 2 * _HEADS)
    return pl.pallas_call(
        _proj_score_body,
        grid=grid,
        in_specs=[
            pl.BlockSpec((1, _TBLK, e), lambda ib, it, iph: (ib, it, 0)),
            pl.BlockSpec((1, _DH, e), lambda ib, it, iph: (iph, 0, 0)),
            pl.BlockSpec((1, _NK, _DH), lambda ib, it, iph: (iph, 0, 0)),
        ],
        out_specs=pl.BlockSpec((1, 1, _TBLK, _NK),
                               lambda ib, it, iph: (ib, iph, it, 0)),
        out_shape=jax.ShapeDtypeStruct((b, 2 * _HEADS, t, _NK), jnp.float32),
    )(x, wq_r, k_r)


_TBLK2 = 512


def _out_proj_body(xh_ref, wo_ref, bo_ref, y_ref):
    ih = pl.program_id(2)
    part = lax.dot_general(xh_ref[0, 0], wo_ref[...],
                           (((1,), (1,)), ((), ())),
                           preferred_element_type=jnp.float32)

    @pl.when(ih == 0)
    def _():
        y_ref[0] = part + bo_ref[0]

    @pl.when(ih != 0)
    def _():
        y_ref[0] += part


def _out_proj(out_heads, wo_r, bo_r):
    b = out_heads.shape[0]
    t = out_heads.shape[2]
    grid = (b, t // _TBLK2, _HEADS)
    return pl.pallas_call(
        _out_proj_body,
        grid=grid,
        in_specs=[
            pl.BlockSpec((1, 1, _TBLK2, _DH), lambda ib, it, ih: (ib, ih, it, 0)),
            pl.BlockSpec((_DIM, _DH), lambda ib, it, ih: (0, ih)),
            pl.BlockSpec((1, _DIM), lambda ib, it, ih: (0, 0)),
        ],
        out_specs=pl.BlockSpec((1, _TBLK2, _DIM), lambda ib, it, ih: (ib, it, 0)),
        out_shape=jax.ShapeDtypeStruct((b, t, _DIM), jnp.float32),
    )(out_heads, wo_r, bo_r)


_CT = 16
_NCH = 512 // _CT


def _sortd(k, v):
    return plsc.sort_key_val(k, v, descending=True)


def _merge16(tv, ti, cv, ci):
    rcv = jnp.flip(cv, 0)
    rci = jnp.flip(ci, 0)
    m = tv >= rcv
    nv = jnp.where(m, tv, rcv)
    ni = jnp.where(m, ti, rci)
    return _sortd(nv, ni)


def _topk128(sd, k_idx, tl):
    tv = sd[k_idx, tl, pl.ds(0, 16)]
    ti = lax.iota(jnp.int32, 16)
    tv, ti = _sortd(tv, ti)
    for kk in range(1, 8):
        cv = sd[k_idx, tl, pl.ds(kk * 16, 16)]
        ci = lax.iota(jnp.int32, 16) + (kk * 16)
        cv, ci = _sortd(cv, ci)
        tv, ti = _merge16(tv, ti, cv, ci)
    return tv, ti


def _pkm_sc(dots_flat, values_flat, n_rows_out):
    mesh = plsc.VectorSubcoreMesh(core_axis_name="c", subcore_axis_name="s")

    @functools.partial(
        pl.kernel,
        out_type=jax.ShapeDtypeStruct((n_rows_out, _DH), jnp.float32),
        mesh=mesh,
        compiler_params=pltpu.CompilerParams(needs_layout_passes=False),
        scratch_types=[
            pltpu.VMEM((4, _CT, _NK), jnp.float32),
            pltpu.VMEM((2 * _CT, _DH), jnp.float32),
            pltpu.VMEM((2, 16, _DH), jnp.float32),
            pltpu.VMEM((2, 16), jnp.int32),
            pltpu.SemaphoreType.DMA,
            pltpu.SemaphoreType.DMA,
        ],
    )
    def body(dots_hbm, values_hbm, out_hbm,
             sd, outb, gb, vib, sem_in, sem_g):
        c = lax.axis_index("c")
        s = lax.axis_index("s")
        b = c
        h = s // 2
        half = s % 2
        row_p = [(b * 16 + h) * 2048, (b * 16 + 8 + h) * 2048]
        tau0 = half * 512
        out_base = (b * 8 + h) * 2048
        vbase = h * (_NK * _NK)

        @pl.loop(0, _NCH)
        def _chunk(ci_):
            t0 = tau0 + ci_ * _CT
            cps = []
            for p in range(2):
                for seg in range(2):
                    rb = row_p[p] + seg * 1024 + t0
                    cps.append(pltpu.async_copy(
                        dots_hbm.at[pl.ds(rb, _CT), :],
                        sd.at[p * 2 + seg], sem_in))
            for cp in cps:
                cp.wait()

            @pl.loop(0, _CT)
            def _row(tl):
                for p in range(2):
                    tv0, ti0 = _topk128(sd, p * 2 + 0, tl)
                    tv1, ti1 = _topk128(sd, p * 2 + 1, tl)
                    rv = tv1 + tv0[0]
                    ri = ti1 + ti0[0] * _NK
                    for i in range(1, 16):
                        cv = tv1 + tv0[i]
                        ci2 = ti1 + ti0[i] * _NK
                        rv, ri = _merge16(rv, ri, cv, ci2)
                    mx = jnp.max(rv)
                    ev = jnp.exp(rv - mx)
                    at = ev / jnp.sum(ev)
                    vib[p, :] = ri + vbase
                    pltpu.async_copy(values_hbm.at[vib.at[p]],
                                     gb.at[p], sem_g).wait()
                    accs = [None] * 16
                    for j in range(16):
                        aj = at[j]
                        for dc in range(16):
                            gv = gb[p, j, pl.ds(dc * 16, 16)]
                            tgv = aj * gv
                            accs[dc] = tgv if j == 0 else accs[dc] + tgv
                    for dc in range(16):
                        outb[2 * tl + p, pl.ds(dc * 16, 16)] = accs[dc]

            pltpu.sync_copy(outb, out_hbm.at[pl.ds(out_base + 2 * t0, 2 * _CT), :])

    return body(dots_flat, values_flat)


def kernel(x, W_q, W_o, b_o, keys, values):
    b, t, e = x.shape
    wq_r = W_q.reshape(2 * _HEADS, _DH, e)
    k_r = jnp.transpose(keys, (2, 0, 1, 3)).reshape(2 * _HEADS, _NK, _DH)
    dots = _proj_score(x, wq_r, k_r)
    dots_flat = dots.reshape(b * 2 * _HEADS * t, _NK)
    values_flat = values.reshape(_HEADS * _NK * _NK, _DH)
    out_heads = _pkm_sc(dots_flat, values_flat, b * _HEADS * t)
    out_heads = out_heads.reshape(b, _HEADS, t, _DH)
    y = _out_proj(out_heads, W_o, b_o.reshape(1, _DIM))
    return y


# speedup vs baseline: 2.8692x; 1.4892x over previous
import functools

import numpy as np

import jax
import jax.numpy as jnp
from jax import lax
from jax.experimental import pallas as pl
from jax.experimental.pallas import tpu as pltpu
from jax.experimental.pallas import tpu_sc as plsc

_DIM = 2048
_HEADS = 8
_NK = 128
_TK = 16
_DH = _DIM // _HEADS

_TBLK = 512


def _proj_score_body(x_ref, wq_ref, k_ref, d_ref):
    xb = x_ref[0]
    wq = wq_ref[0]
    kk = k_ref[0]
    q = lax.dot_general(xb, wq, (((1,), (1,)), ((), ())),
                        preferred_element_type=jnp.float32)
    d = lax.dot_general(q, kk, (((1,), (1,)), ((), ())),
                        preferred_element_type=jnp.float32)
    d_ref[0, 0] = d


def _proj_score(x, wq_r, k_r):
    b, t, e = x.shape
    grid = (b, t // _TBLK, 2 * _HEADS)
    return pl.pallas_call(
        _proj_score_body,
        grid=grid,
        in_specs=[
            pl.BlockSpec((1, _TBLK, e), lambda ib, it, iph: (ib, it, 0)),
            pl.BlockSpec((1, _DH, e), lambda ib, it, iph: (iph, 0, 0)),
            pl.BlockSpec((1, _NK, _DH), lambda ib, it, iph: (iph, 0, 0)),
        ],
        out_specs=pl.BlockSpec((1, 1, _TBLK, _NK),
                               lambda ib, it, iph: (ib, iph, it, 0)),
        out_shape=jax.ShapeDtypeStruct((b, 2 * _HEADS, t, _NK), jnp.float32),
    )(x, wq_r, k_r)


_TBLK2 = 512


def _out_proj_body(xh_ref, wo_ref, bo_ref, y_ref):
    ih = pl.program_id(2)
    part = lax.dot_general(xh_ref[0, 0], wo_ref[...],
                           (((1,), (1,)), ((), ())),
                           preferred_element_type=jnp.float32)

    @pl.when(ih == 0)
    def _():
        y_ref[0] = part + bo_ref[0]

    @pl.when(ih != 0)
    def _():
        y_ref[0] += part


def _out_proj(out_heads, wo_r, bo_r):
    b = out_heads.shape[0]
    t = out_heads.shape[2]
    grid = (b, t // _TBLK2, _HEADS)
    return pl.pallas_call(
        _out_proj_body,
        grid=grid,
        in_specs=[
            pl.BlockSpec((1, 1, _TBLK2, _DH), lambda ib, it, ih: (ib, ih, it, 0)),
            pl.BlockSpec((_DIM, _DH), lambda ib, it, ih: (0, ih)),
            pl.BlockSpec((1, _DIM), lambda ib, it, ih: (0, 0)),
        ],
        out_specs=pl.BlockSpec((1, _TBLK2, _DIM), lambda ib, it, ih: (ib, it, 0)),
        out_shape=jax.ShapeDtypeStruct((b, t, _DIM), jnp.float32),
    )(out_heads, wo_r, bo_r)


_CT = 16
_NCH = 512 // _CT

_CAND = sorted(((i + 1) * (j + 1), i, j)
               for i in range(16) for j in range(16))[:64]
_CAND_I = tuple(i for _, i, _j in _CAND)
_CAND_J = tuple(j for _, _i, j in _CAND)


def _sortd(k, v):
    return plsc.sort_key_val(k, v, descending=True)


def _tophalf(tv, ti, cv, ci):
    rcv = jnp.flip(cv, 0)
    rci = jnp.flip(ci, 0)
    m = tv >= rcv
    return jnp.where(m, tv, rcv), jnp.where(m, ti, rci)


def _merge16(tv, ti, cv, ci):
    nv, ni = _tophalf(tv, ti, cv, ci)
    return _sortd(nv, ni)


def _topk128(sd, par, k_idx, tl):
    lvl = []
    for kk in range(8):
        cv = sd[par, k_idx, tl, pl.ds(kk * 16, 16)]
        ci = lax.iota(jnp.int32, 16) + (kk * 16)
        lvl.append(_sortd(cv, ci))
    while len(lvl) > 1:
        nxt = []
        for a in range(0, len(lvl), 2):
            nxt.append(_merge16(*lvl[a], *lvl[a + 1]))
        lvl = nxt
    return lvl[0]


def _take16(v, idx):
    return v.at[idx].get(mode="promise_in_bounds", unique_indices=False)


def _combine(tv0, ti0, tv1, ti1, civ, cjv):
    del civ, cjv
    lvl = []
    for i in range(16):
        lvl.append((tv1 + tv0[i], ti1 + ti0[i] * _NK))
    while len(lvl) > 2:
        nxt = []
        for a in range(0, len(lvl), 2):
            nxt.append(_merge16(*lvl[a], *lvl[a + 1]))
        lvl = nxt
    return _tophalf(*lvl[0], *lvl[1])


def _pkm_sc(dots_flat, values_flat, cand_ij, n_rows_out):
    mesh = plsc.VectorSubcoreMesh(core_axis_name="c", subcore_axis_name="s",
                                  num_cores=2, num_subcores=16)

    @functools.partial(
        pl.kernel,
        out_type=jax.ShapeDtypeStruct((n_rows_out, _DH), jnp.float32),
        mesh=mesh,
        compiler_params=pltpu.CompilerParams(needs_layout_passes=False),
        scratch_types=[
            pltpu.VMEM((2, 4, _CT, _NK), jnp.float32),
            pltpu.VMEM((2 * _CT, _DH), jnp.float32),
            pltpu.VMEM((2, 32, _DH), jnp.float32),
            pltpu.VMEM((2, 32), jnp.int32),
            pltpu.VMEM((2, 4, 16), jnp.int32),
            pltpu.SemaphoreType.DMA,
            pltpu.SemaphoreType.DMA,
            pltpu.SemaphoreType.DMA,
        ],
    )
    def body(dots_hbm, values_hbm, cand_hbm, out_hbm,
             sd, outb, gb, vib, candv, sem_in, sem_g0, sem_g1):
        pltpu.sync_copy(cand_hbm, candv)
        c = lax.axis_index("c")
        s = lax.axis_index("s")
        b = c
        h = s // 2
        half = s % 2
        row_p = [(b * 16 + h) * 2048, (b * 16 + 8 + h) * 2048]
        tau0 = half * 512
        out_base = (b * 8 + h) * 2048
        vbase = h * (_NK * _NK)
        sem_g = [sem_g0, sem_g1]

        def issue_stage(ci_, par):
            t0 = tau0 + ci_ * _CT
            for p in range(2):
                for seg in range(2):
                    rb = row_p[p] + seg * 1024 + t0
                    pltpu.async_copy(dots_hbm.at[pl.ds(rb, _CT), :],
                                     sd.at[par, p * 2 + seg], sem_in)

        issue_stage(0, 0)

        @pl.loop(0, _NCH)
        def _chunk(ci_):
            par = lax.rem(ci_, 2)
            t0 = tau0 + ci_ * _CT
            for k in range(4):
                pltpu.make_async_copy(dots_hbm.at[pl.ds(0, _CT), :],
                                      sd.at[par, k], sem_in).wait()

            @pl.when(ci_ + 1 < _NCH)
            def _():
                issue_stage(ci_ + 1, 1 - par)

            @pl.loop(0, _CT // 2)
            def _pair(tp):
                rows = ((2 * tp, 0), (2 * tp + 1, 1))
                atts = [[None, None], [None, None]]
                for tl, bufi in rows:
                    for p in range(2):
                        tv0, ti0 = _topk128(sd, par, p * 2 + 0, tl)
                        tv1, ti1 = _topk128(sd, par, p * 2 + 1, tl)
                        rv, ri = _combine(tv0, ti0, tv1, ti1,
                                          candv.at[0], candv.at[1])
                        mx = jnp.max(rv)
                        ev = jnp.exp(rv - mx)
                        atts[bufi][p] = ev / jnp.sum(ev)
                        vib[bufi, pl.ds(p * 16, 16)] = ri + vbase
                    pltpu.async_copy(values_hbm.at[vib.at[bufi]],
                                     gb.at[bufi], sem_g[bufi])
                for tl, bufi in rows:
                    pltpu.make_async_copy(values_hbm.at[vib.at[bufi]],
                                          gb.at[bufi], sem_g[bufi]).wait()
                    for p in range(2):
                        at = atts[bufi][p]
                        accs = [None] * 16
                        for j in range(16):
                            aj = at[j]
                            for dc in range(16):
                                gv = gb[bufi, p * 16 + j, pl.ds(dc * 16, 16)]
                                tgv = aj * gv
                                accs[dc] = tgv if j == 0 else accs[dc] + tgv
                        for dc in range(16):
                            outb[2 * tl + p, pl.ds(dc * 16, 16)] = accs[dc]

            pltpu.sync_copy(outb, out_hbm.at[pl.ds(out_base + 2 * t0, 2 * _CT), :])

    return body(dots_flat, values_flat, cand_ij)


def kernel(x, W_q, W_o, b_o, keys, values):
    b, t, e = x.shape
    wq_r = W_q.reshape(2 * _HEADS, _DH, e)
    k_r = jnp.transpose(keys, (2, 0, 1, 3)).reshape(2 * _HEADS, _NK, _DH)
    dots = _proj_score(x, wq_r, k_r)
    dots_flat = dots.reshape(b * 2 * _HEADS * t, _NK)
    values_flat = values.reshape(_HEADS * _NK * _NK, _DH)
    cand_ij = jnp.asarray(
        np.stack([np.reshape(_CAND_I, (4, 16)), np.reshape(_CAND_J, (4, 16))]),
        dtype=jnp.int32)
    out_heads = _pkm_sc(dots_flat, values_flat, cand_ij,
                        b * _HEADS * t)
    out_heads = out_heads.reshape(b, _HEADS, t, _DH)
    y = _out_proj(out_heads, W_o, b_o.reshape(1, _DIM))
    return y
